# Initial kernel scaffold; baseline (speedup 1.0000x reference)
import jax
import jax.numpy as jnp
from jax.experimental import pallas as pl


def kernel(tokens, token_table, pos_table):
    raise NotImplementedError("write your pallas kernel here")


# speedup vs baseline: 2.3711x; 2.3711x over previous
import functools

import jax
import jax.numpy as jnp
from jax import lax
from jax.experimental import pallas as pl
from jax.experimental.pallas import tpu as pltpu
from jax.experimental.pallas import tpu_sc as plsc

NC = 2
NS = 16
NW = NC * NS


def _make_sc_gather(n_rows: int, seq_len: int, dim: int):
    assert n_rows % (NW * seq_len) == 0
    rows_per_w = n_rows // NW
    nchunks = rows_per_w // seq_len
    mesh = plsc.VectorSubcoreMesh(core_axis_name="c", subcore_axis_name="s")

    @functools.partial(
        pl.kernel,
        out_type=jax.ShapeDtypeStruct((n_rows, dim), jnp.float32),
        mesh=mesh,
        scratch_types=[
            pltpu.VMEM((seq_len,), jnp.int32),
            pltpu.VMEM((seq_len, dim), jnp.float32),
            pltpu.VMEM((seq_len, dim), jnp.float32),
            pltpu.SemaphoreType.DMA,
        ],
        compiler_params=pltpu.CompilerParams(use_tc_tiling_on_sc=False),
    )
    def sc_kernel(tok_hbm, table_hbm, pos_hbm, out_hbm,
                  idx_v, rows_v, pos_v, sem):
        wid = lax.axis_index("s") * NC + lax.axis_index("c")
        base0 = wid * rows_per_w
        pltpu.sync_copy(pos_hbm, pos_v)

        def chunk(g, carry):
            base = base0 + g * seq_len
            pltpu.sync_copy(tok_hbm.at[pl.ds(base, seq_len)], idx_v)
            pltpu.async_copy(table_hbm.at[idx_v], rows_v, sem).wait()

            def radd(r, c):
                for d in range(dim // 16):
                    sl = pl.ds(d * 16, 16)
                    rows_v[r, sl] = rows_v[r, sl] + pos_v[r, sl]
                return c

            lax.fori_loop(0, seq_len, radd, 0)
            pltpu.sync_copy(rows_v, out_hbm.at[pl.ds(base, seq_len)])
            return carry

        lax.fori_loop(0, nchunks, chunk, 0)

    return sc_kernel


def kernel(tokens, token_table, pos_table):
    batch, seq_len = tokens.shape
    dim = token_table.shape[1]
    n_rows = batch * seq_len
    idx_flat = tokens.reshape(n_rows).astype(jnp.int32)
    out = _make_sc_gather(n_rows, seq_len, dim)(
        idx_flat, token_table, pos_table)
    return out.reshape(batch, seq_len, dim)


# speedup vs baseline: 2.7652x; 1.1662x over previous
import functools

import jax
import jax.numpy as jnp
from jax import lax
from jax.experimental import pallas as pl
from jax.experimental.pallas import tpu as pltpu
from jax.experimental.pallas import tpu_sc as plsc

NC = 2
NS = 16
NW = NC * NS
NBUF = 4


def _make_sc_gather(n_rows: int, seq_len: int, dim: int):
    assert n_rows % (NW * seq_len) == 0
    rows_per_w = n_rows // NW
    nchunks = rows_per_w // seq_len
    assert nchunks % NBUF == 0 and nchunks >= 2 * NBUF
    mesh = plsc.VectorSubcoreMesh(core_axis_name="c", subcore_axis_name="s")

    @functools.partial(
        pl.kernel,
        out_type=jax.ShapeDtypeStruct((n_rows, dim), jnp.float32),
        mesh=mesh,
        scratch_types=[
            pltpu.VMEM((rows_per_w,), jnp.int32),
            pltpu.VMEM((seq_len, dim), jnp.float32),
            pltpu.VMEM((seq_len, dim), jnp.float32),
            pltpu.VMEM((seq_len, dim), jnp.float32),
            pltpu.VMEM((seq_len, dim), jnp.float32),
            pltpu.VMEM((seq_len, dim), jnp.float32),
            pltpu.SemaphoreType.DMA,
            pltpu.SemaphoreType.DMA,
        ],
        compiler_params=pltpu.CompilerParams(use_tc_tiling_on_sc=False),
    )
    def sc_kernel(tok_hbm, table_hbm, pos_hbm, out_hbm,
                  idx_all, r0, r1, r2, r3, pos_v, gsem, osem):
        rows = [r0, r1, r2, r3]
        wid = lax.axis_index("s") * NC + lax.axis_index("c")
        base0 = wid * rows_per_w
        pltpu.sync_copy(pos_hbm, pos_v)
        pltpu.sync_copy(tok_hbm.at[pl.ds(base0, rows_per_w)], idx_all)

        def idx_slice(g):
            return idx_all.at[pl.ds(g * seq_len, seq_len)]

        def issue_gather(g, b):
            pltpu.async_copy(table_hbm.at[idx_slice(g)], rows[b], gsem)

        def wait_gather(g, b):
            pltpu.make_async_copy(table_hbm.at[idx_slice(g)], rows[b],
                                  gsem).wait()

        def drain_out(b):
            pltpu.make_async_copy(rows[b], out_hbm.at[pl.ds(base0, seq_len)],
                                  osem).wait()

        def start_out(g, b):
            pltpu.async_copy(
                rows[b], out_hbm.at[pl.ds(base0 + g * seq_len, seq_len)], osem)

        def alu_add(b):
            rb = rows[b]
            runroll = 4

            def radd(i, c):
                r0 = i * runroll
                for u in range(runroll):
                    for d in range(dim // 16):
                        sl = pl.ds(d * 16, 16)
                        rb[r0 + u, sl] = rb[r0 + u, sl] + pos_v[r0 + u, sl]
                return c

            lax.fori_loop(0, seq_len // runroll, radd, 0)

        def half_iter(g, b, drain, prefetch):
            p = (b + NBUF - 1) % NBUF
            if drain:
                drain_out(p)
            if prefetch:
                issue_gather(g + NBUF - 1, p)
            wait_gather(g, b)
            alu_add(b)
            start_out(g, b)

        for b in range(NBUF - 1):
            issue_gather(b, b)

        half_iter(0, 0, False, True)
        for b in range(1, NBUF):
            half_iter(b, b, True, True)

        @pl.loop(1, nchunks // NBUF - 1)
        def _(go):
            g0 = go * NBUF
            for b in range(NBUF):
                half_iter(g0 + b, b, True, True)

        g0 = nchunks - NBUF
        half_iter(g0, 0, True, True)
        for b in range(1, NBUF):
            half_iter(g0 + b, b, True, False)
        drain_out(NBUF - 1)

    return sc_kernel


def kernel(tokens, token_table, pos_table):
    batch, seq_len = tokens.shape
    dim = token_table.shape[1]
    n_rows = batch * seq_len
    idx_flat = tokens.reshape(n_rows).astype(jnp.int32)
    out = _make_sc_gather(n_rows, seq_len, dim)(
        idx_flat, token_table, pos_table)
    return out.reshape(batch, seq_len, dim)
